# Initial kernel scaffold; baseline (speedup 1.0000x reference)
import jax
import jax.numpy as jnp
from jax.experimental import pallas as pl


def kernel(x_s, x_t, params, edge_index_s, edge_index_t, xs_batch, xt_batch):
    raise NotImplementedError("write your pallas kernel here")


# speedup vs baseline: 26.2573x; 26.2573x over previous
import functools

import jax
import jax.numpy as jnp
from jax import lax
from jax.experimental import pallas as pl
from jax.experimental.pallas import tpu as pltpu
from jax.experimental.pallas import tpu_sc as plsc

NC = 2
NS = 16
NW = NC * NS
LANE = 128


def _sc_mesh():
    return plsc.VectorSubcoreMesh(core_axis_name="c", subcore_axis_name="s")


def _make_sc_degree(n_pad, groups):
    rpt = n_pad // NS

    @functools.partial(
        pl.kernel,
        out_type=jax.ShapeDtypeStruct((NC, n_pad), jnp.float32),
        mesh=_sc_mesh(),
        scratch_types=[
            pltpu.VMEM((groups, LANE), jnp.int32),
            pltpu.VMEM((LANE,), jnp.float32),
            pltpu.VMEM((LANE,), jnp.float32),
            pltpu.VMEM_SHARED((n_pad,), jnp.float32),
        ],
    )
    def deg_kernel(col_hbm, out_hbm, cidx, ones_v, zbuf, acc):
        cid = lax.axis_index("c")
        sid = lax.axis_index("s")
        w = cid * NS + sid
        base = sid * rpt
        for i in range(LANE // 16):
            ones_v[pl.ds(i * 16, 16)] = jnp.ones((16,), jnp.float32)
            zbuf[pl.ds(i * 16, 16)] = jnp.zeros((16,), jnp.float32)
        off = 0
        while off < rpt:
            sz = min(LANE, rpt - off)
            pltpu.sync_copy(zbuf.at[pl.ds(0, sz)], acc.at[pl.ds(base + off, sz)])
            off += sz
        plsc.subcore_barrier()
        pltpu.sync_copy(col_hbm.at[w], cidx)

        def body(g, carry):
            pltpu.sync_copy(ones_v, acc.at[cidx.at[g]], add=True)
            return carry

        lax.fori_loop(0, groups, body, 0)
        plsc.subcore_barrier()
        off = 0
        while off < rpt:
            sz = min(LANE, rpt - off)
            pltpu.sync_copy(acc.at[pl.ds(base + off, sz)],
                            out_hbm.at[cid, pl.ds(base + off, sz)])
            off += sz

    return deg_kernel


def _make_sc_scatter(n_pad, groups, dout):
    rpt = n_pad // NS

    @functools.partial(
        pl.kernel,
        out_type=jax.ShapeDtypeStruct((NC, n_pad, dout), jnp.float32),
        mesh=_sc_mesh(),
        scratch_types=[
            pltpu.VMEM((groups, LANE), jnp.int32),
            pltpu.VMEM((groups, LANE), jnp.int32),
            pltpu.VMEM((LANE, dout), jnp.float32),
            pltpu.VMEM((LANE, dout), jnp.float32),
            pltpu.VMEM_SHARED((n_pad, dout), jnp.float32),
            pltpu.SemaphoreType.DMA,
        ],
        compiler_params=pltpu.CompilerParams(use_tc_tiling_on_sc=False),
    )
    def scatter_kernel(g_hbm, row_hbm, col_hbm, zero_hbm, out_hbm,
                       ridx, cidx, rows, zbuf, acc, sem):
        cid = lax.axis_index("c")
        sid = lax.axis_index("s")
        w = cid * NS + sid
        base = sid * rpt
        pltpu.sync_copy(zero_hbm, zbuf)
        off = 0
        while off < rpt:
            sz = min(LANE, rpt - off)
            pltpu.sync_copy(zbuf.at[pl.ds(0, sz)], acc.at[pl.ds(base + off, sz)])
            off += sz
        plsc.subcore_barrier()
        pltpu.sync_copy(row_hbm.at[w], ridx)
        pltpu.sync_copy(col_hbm.at[w], cidx)

        def body(g, carry):
            pltpu.async_copy(g_hbm.at[ridx.at[g]], rows, sem).wait()
            pltpu.sync_copy(rows, acc.at[cidx.at[g]], add=True)
            return carry

        lax.fori_loop(0, groups, body, 0)
        plsc.subcore_barrier()
        off = 0
        while off < rpt:
            sz = min(LANE, rpt - off)
            pltpu.sync_copy(acc.at[pl.ds(base + off, sz)],
                            out_hbm.at[cid, pl.ds(base + off, sz)])
            off += sz

    return scatter_kernel


def _tc_layer1_body(x_ref, w_ref, deg_ref, g_ref, dinv_ref):
    deg = 1.0 + deg_ref[0] + deg_ref[1]
    dinv = lax.rsqrt(deg)
    dinv_ref[...] = dinv
    h = jnp.dot(x_ref[...], w_ref[...], preferred_element_type=jnp.float32)
    g_ref[...] = dinv * h


def _tc_mid_body(p_ref, g_ref, dinv_ref, b_ref, gw_ref, gb_ref, ga_ref,
                 w_ref, out_ref):
    dinv = dinv_ref[...]
    s = p_ref[0] + p_ref[1] + g_ref[...]
    y = jnp.maximum(dinv * s + b_ref[...], 0.0)
    a = ga_ref[...]
    m = jnp.mean(y, axis=0, keepdims=True)
    o = y - a * m
    var = jnp.mean(o * o, axis=0, keepdims=True)
    xg = gw_ref[...] * o / jnp.sqrt(var + 1e-5) + gb_ref[...]
    h = jnp.dot(xg, w_ref[...], preferred_element_type=jnp.float32)
    out_ref[...] = dinv * h


def _tc_head_body(p_ref, g_ref, dinv_ref, b_ref, gw_ref, gb_ref, ga_ref,
                  batch_ref, wl1_ref, bl1_ref, bn1w_ref, bn1b_ref,
                  wl2_ref, bl2_ref, bn2w_ref, bn2b_ref, wl3_ref, bl3_ref,
                  h_ref, sig_ref, *, num_graphs):
    s = p_ref[0] + p_ref[1] + g_ref[...]
    y = jnp.maximum(dinv_ref[...] * s + b_ref[...], 0.0)
    a = ga_ref[...]
    m = jnp.mean(y, axis=0, keepdims=True)
    o = y - a * m
    var = jnp.mean(o * o, axis=0, keepdims=True)
    x3 = gw_ref[...] * o / jnp.sqrt(var + 1e-5) + gb_ref[...]

    n = x3.shape[0]
    gid = lax.broadcasted_iota(jnp.int32, (num_graphs, n), 0)
    oh = (gid == batch_ref[...]).astype(jnp.float32)
    ssum = jnp.dot(oh, x3, preferred_element_type=jnp.float32)
    cnt = jnp.sum(oh, axis=1, keepdims=True)
    xs = ssum / jnp.maximum(cnt, 1.0)

    h1 = jnp.dot(xs, wl1_ref[...], preferred_element_type=jnp.float32) + bl1_ref[...]
    m1 = jnp.mean(h1, axis=0, keepdims=True)
    v1 = jnp.mean((h1 - m1) ** 2, axis=0, keepdims=True)
    h1 = bn1w_ref[...] * (h1 - m1) / jnp.sqrt(v1 + 1e-5) + bn1b_ref[...]

    h2 = jnp.dot(h1, wl2_ref[...], preferred_element_type=jnp.float32) + bl2_ref[...]
    m2 = jnp.mean(h2, axis=0, keepdims=True)
    v2 = jnp.mean((h2 - m2) ** 2, axis=0, keepdims=True)
    h2 = bn2w_ref[...] * (h2 - m2) / jnp.sqrt(v2 + 1e-5) + bn2b_ref[...]

    h = jnp.dot(h2, wl3_ref[...], preferred_element_type=jnp.float32) + bl3_ref[...]
    h_ref[...] = h
    sig_ref[...] = jax.nn.sigmoid(h)


def _row(v):
    return v.reshape(1, -1)


def kernel(x_s, x_t, params, edge_index_s, edge_index_t, xs_batch, xt_batch):
    p = params
    n, d_in = x_s.shape
    e = edge_index_s.shape[1]
    num_graphs = 64

    assert e % NW == 0
    ept = e // NW
    groups = -(-ept // LANE)
    ept_pad = groups * LANE
    pad = ept_pad - ept
    n_pad = ((n + NS * 8 - 1) // (NS * 8)) * (NS * 8)
    if n_pad <= n + pad:
        n_pad += NS * 8

    row = edge_index_s[0].astype(jnp.int32).reshape(NW, ept)
    col = edge_index_s[1].astype(jnp.int32).reshape(NW, ept)
    pr = ((jnp.arange(pad, dtype=jnp.int32) * 89) % n)
    pc = n + (jnp.arange(pad, dtype=jnp.int32) % (n_pad - n))
    rowp = jnp.concatenate(
        [row, jnp.broadcast_to(pr, (NW, pad))], axis=1).reshape(NW, groups, LANE)
    colp = jnp.concatenate(
        [col, jnp.broadcast_to(pc, (NW, pad))], axis=1).reshape(NW, groups, LANE)

    deg2 = _make_sc_degree(n_pad, groups)(colp)
    deg2 = deg2[:, :n].reshape(NC, n, 1)

    dims = [d_in, d_in // 2, d_in // 4, d_in // 8]

    g1, dinv = pl.pallas_call(
        _tc_layer1_body,
        out_shape=(jax.ShapeDtypeStruct((n, dims[1]), jnp.float32),
                   jax.ShapeDtypeStruct((n, 1), jnp.float32)),
    )(x_s, p['Ws1'], deg2)

    def sc_pass(g, dout):
        z = jnp.zeros((LANE, dout), jnp.float32)
        part = _make_sc_scatter(n_pad, groups, dout)(g, rowp, colp, z)
        return part[:, :n]

    def tc_mid(part, g, b, gw, gb, ga, w_next, dnext):
        return pl.pallas_call(
            _tc_mid_body,
            out_shape=jax.ShapeDtypeStruct((n, dnext), jnp.float32),
        )(part, g, dinv, _row(b), _row(gw), _row(gb), _row(ga), w_next)

    p1 = sc_pass(g1, dims[1])
    g2 = tc_mid(p1, g1, p['bs1'], p['gn1_w'], p['gn1_b'], p['gn1_a'],
                p['Ws2'], dims[2])
    p2 = sc_pass(g2, dims[2])
    g3 = tc_mid(p2, g2, p['bs2'], p['gn2_w'], p['gn2_b'], p['gn2_a'],
                p['Ws3'], dims[3])
    p3 = sc_pass(g3, dims[3])

    out_dim = p['Wl3'].shape[1]
    h, sig = pl.pallas_call(
        functools.partial(_tc_head_body, num_graphs=num_graphs),
        out_shape=(jax.ShapeDtypeStruct((num_graphs, out_dim), jnp.float32),
                   jax.ShapeDtypeStruct((num_graphs, out_dim), jnp.float32)),
    )(p3, g3, dinv, _row(p['bs3']), _row(p['gn3_w']), _row(p['gn3_b']),
      _row(p['gn3_a']), _row(xs_batch.astype(jnp.int32)),
      p['Wl1'], _row(p['bl1']), _row(p['bn1_w']), _row(p['bn1_b']),
      p['Wl2'], _row(p['bl2']), _row(p['bn2_w']), _row(p['bn2_b']),
      p['Wl3'], _row(p['bl3']))
    return (h, sig)


# speedup vs baseline: 34.0925x; 1.2984x over previous
import functools

import jax
import jax.numpy as jnp
from jax import lax
from jax.experimental import pallas as pl
from jax.experimental.pallas import tpu as pltpu
from jax.experimental.pallas import tpu_sc as plsc

NC = 2
NS = 16
NW = NC * NS
LANE = 128


def _sc_mesh():
    return plsc.VectorSubcoreMesh(core_axis_name="c", subcore_axis_name="s")


def _make_sc_degree(n_pad, groups):
    rpt = n_pad // NS

    @functools.partial(
        pl.kernel,
        out_type=jax.ShapeDtypeStruct((NC, n_pad), jnp.float32),
        mesh=_sc_mesh(),
        scratch_types=[
            pltpu.VMEM((groups, LANE), jnp.int32),
            pltpu.VMEM((LANE,), jnp.float32),
            pltpu.VMEM((LANE,), jnp.float32),
            pltpu.VMEM_SHARED((n_pad,), jnp.float32),
        ],
    )
    def deg_kernel(col_hbm, out_hbm, cidx, ones_v, zbuf, acc):
        cid = lax.axis_index("c")
        sid = lax.axis_index("s")
        w = cid * NS + sid
        base = sid * rpt
        for i in range(LANE // 16):
            ones_v[pl.ds(i * 16, 16)] = jnp.ones((16,), jnp.float32)
            zbuf[pl.ds(i * 16, 16)] = jnp.zeros((16,), jnp.float32)
        off = 0
        while off < rpt:
            sz = min(LANE, rpt - off)
            pltpu.sync_copy(zbuf.at[pl.ds(0, sz)], acc.at[pl.ds(base + off, sz)])
            off += sz
        plsc.subcore_barrier()
        pltpu.sync_copy(col_hbm.at[w], cidx)

        def body(g, carry):
            pltpu.sync_copy(ones_v, acc.at[cidx.at[g]], add=True)
            return carry

        lax.fori_loop(0, groups, body, 0)
        plsc.subcore_barrier()
        off = 0
        while off < rpt:
            sz = min(LANE, rpt - off)
            pltpu.sync_copy(acc.at[pl.ds(base + off, sz)],
                            out_hbm.at[cid, pl.ds(base + off, sz)])
            off += sz

    return deg_kernel


def _make_sc_scatter(n_pad, groups, dout):
    rpt = n_pad // NS

    @functools.partial(
        pl.kernel,
        out_type=jax.ShapeDtypeStruct((NC, n_pad, dout), jnp.float32),
        mesh=_sc_mesh(),
        scratch_types=[
            pltpu.VMEM((groups, LANE), jnp.int32),
            pltpu.VMEM((groups, LANE), jnp.int32),
            pltpu.VMEM((LANE, dout), jnp.float32),
            pltpu.VMEM((LANE, dout), jnp.float32),
            pltpu.VMEM((LANE, dout), jnp.float32),
            pltpu.VMEM_SHARED((n_pad, dout), jnp.float32),
            pltpu.SemaphoreType.DMA,
            pltpu.SemaphoreType.DMA,
        ],
        compiler_params=pltpu.CompilerParams(use_tc_tiling_on_sc=False),
    )
    def scatter_kernel(g_hbm, row_hbm, col_hbm, zero_hbm, out_hbm,
                       ridx, cidx, rows_a, rows_b, zbuf, acc, sem_a, sem_b):
        cid = lax.axis_index("c")
        sid = lax.axis_index("s")
        w = cid * NS + sid
        base = sid * rpt
        pltpu.sync_copy(zero_hbm, zbuf)
        off = 0
        while off < rpt:
            sz = min(LANE, rpt - off)
            pltpu.sync_copy(zbuf.at[pl.ds(0, sz)], acc.at[pl.ds(base + off, sz)])
            off += sz
        plsc.subcore_barrier()
        pltpu.sync_copy(row_hbm.at[w], ridx)
        pltpu.sync_copy(col_hbm.at[w], cidx)

        assert groups % 2 == 1
        pairs = groups // 2
        pltpu.async_copy(g_hbm.at[ridx.at[0]], rows_a, sem_a)

        def body(i, carry):
            g0 = 2 * i
            pltpu.async_copy(g_hbm.at[ridx.at[g0 + 1]], rows_b, sem_b)
            pltpu.make_async_copy(g_hbm.at[ridx.at[g0]], rows_a, sem_a).wait()
            pltpu.sync_copy(rows_a, acc.at[cidx.at[g0]], add=True)
            pltpu.async_copy(g_hbm.at[ridx.at[g0 + 2]], rows_a, sem_a)
            pltpu.make_async_copy(g_hbm.at[ridx.at[g0 + 1]], rows_b, sem_b).wait()
            pltpu.sync_copy(rows_b, acc.at[cidx.at[g0 + 1]], add=True)
            return carry

        lax.fori_loop(0, pairs, body, 0)
        if groups % 2:
            g_last = groups - 1
            pltpu.make_async_copy(g_hbm.at[ridx.at[g_last]], rows_a, sem_a).wait()
            pltpu.sync_copy(rows_a, acc.at[cidx.at[g_last]], add=True)
        plsc.subcore_barrier()
        off = 0
        while off < rpt:
            sz = min(LANE, rpt - off)
            pltpu.sync_copy(acc.at[pl.ds(base + off, sz)],
                            out_hbm.at[cid, pl.ds(base + off, sz)])
            off += sz

    return scatter_kernel


def _tc_layer1_body(x_ref, w_ref, deg_ref, g_ref, dinv_ref):
    deg = 1.0 + deg_ref[0] + deg_ref[1]
    dinv = lax.rsqrt(deg)
    dinv_ref[...] = dinv
    h = jnp.dot(x_ref[...], w_ref[...], preferred_element_type=jnp.float32, precision=lax.Precision.HIGHEST)
    g_ref[...] = dinv * h


def _tc_mid_body(p_ref, g_ref, dinv_ref, b_ref, gw_ref, gb_ref, ga_ref,
                 w_ref, out_ref):
    dinv = dinv_ref[...]
    s = p_ref[0] + p_ref[1] + g_ref[...]
    y = jnp.maximum(dinv * s + b_ref[...], 0.0)
    a = ga_ref[...]
    m = jnp.mean(y, axis=0, keepdims=True)
    o = y - a * m
    var = jnp.mean(o * o, axis=0, keepdims=True)
    xg = gw_ref[...] * o / jnp.sqrt(var + 1e-5) + gb_ref[...]
    h = jnp.dot(xg, w_ref[...], preferred_element_type=jnp.float32, precision=lax.Precision.HIGHEST)
    out_ref[...] = dinv * h


def _tc_head_body(p_ref, g_ref, dinv_ref, b_ref, gw_ref, gb_ref, ga_ref,
                  batch_ref, wl1_ref, bl1_ref, bn1w_ref, bn1b_ref,
                  wl2_ref, bl2_ref, bn2w_ref, bn2b_ref, wl3_ref, bl3_ref,
                  h_ref, sig_ref, *, num_graphs):
    s = p_ref[0] + p_ref[1] + g_ref[...]
    y = jnp.maximum(dinv_ref[...] * s + b_ref[...], 0.0)
    a = ga_ref[...]
    m = jnp.mean(y, axis=0, keepdims=True)
    o = y - a * m
    var = jnp.mean(o * o, axis=0, keepdims=True)
    x3 = gw_ref[...] * o / jnp.sqrt(var + 1e-5) + gb_ref[...]

    n = x3.shape[0]
    gid = lax.broadcasted_iota(jnp.int32, (num_graphs, n), 0)
    oh = (gid == batch_ref[...]).astype(jnp.float32)
    ssum = jnp.dot(oh, x3, preferred_element_type=jnp.float32, precision=lax.Precision.HIGHEST)
    cnt = jnp.sum(oh, axis=1, keepdims=True)
    xs = ssum / jnp.maximum(cnt, 1.0)

    h1 = jnp.dot(xs, wl1_ref[...], preferred_element_type=jnp.float32, precision=lax.Precision.HIGHEST) + bl1_ref[...]
    m1 = jnp.mean(h1, axis=0, keepdims=True)
    v1 = jnp.mean((h1 - m1) ** 2, axis=0, keepdims=True)
    h1 = bn1w_ref[...] * (h1 - m1) / jnp.sqrt(v1 + 1e-5) + bn1b_ref[...]

    h2 = jnp.dot(h1, wl2_ref[...], preferred_element_type=jnp.float32, precision=lax.Precision.HIGHEST) + bl2_ref[...]
    m2 = jnp.mean(h2, axis=0, keepdims=True)
    v2 = jnp.mean((h2 - m2) ** 2, axis=0, keepdims=True)
    h2 = bn2w_ref[...] * (h2 - m2) / jnp.sqrt(v2 + 1e-5) + bn2b_ref[...]

    h = jnp.dot(h2, wl3_ref[...], preferred_element_type=jnp.float32, precision=lax.Precision.HIGHEST) + bl3_ref[...]
    h_ref[...] = h
    sig_ref[...] = jax.nn.sigmoid(h)


def _row(v):
    return v.reshape(1, -1)


def kernel(x_s, x_t, params, edge_index_s, edge_index_t, xs_batch, xt_batch):
    p = params
    n, d_in = x_s.shape
    e = edge_index_s.shape[1]
    num_graphs = 64

    assert e % NW == 0
    ept = e // NW
    groups = -(-ept // LANE)
    ept_pad = groups * LANE
    pad = ept_pad - ept
    n_pad = ((n + NS * 8 - 1) // (NS * 8)) * (NS * 8)
    if n_pad <= n + pad:
        n_pad += NS * 8

    row = edge_index_s[0].astype(jnp.int32).reshape(NW, ept)
    col = edge_index_s[1].astype(jnp.int32).reshape(NW, ept)
    pr = ((jnp.arange(pad, dtype=jnp.int32) * 89) % n)
    pc = n + (jnp.arange(pad, dtype=jnp.int32) % (n_pad - n))
    rowp = jnp.concatenate(
        [row, jnp.broadcast_to(pr, (NW, pad))], axis=1).reshape(NW, groups, LANE)
    colp = jnp.concatenate(
        [col, jnp.broadcast_to(pc, (NW, pad))], axis=1).reshape(NW, groups, LANE)

    deg2 = _make_sc_degree(n_pad, groups)(colp)
    deg2 = deg2[:, :n].reshape(NC, n, 1)

    dims = [d_in, d_in // 2, d_in // 4, d_in // 8]

    g1, dinv = pl.pallas_call(
        _tc_layer1_body,
        out_shape=(jax.ShapeDtypeStruct((n, dims[1]), jnp.float32),
                   jax.ShapeDtypeStruct((n, 1), jnp.float32)),
    )(x_s, p['Ws1'], deg2)

    def sc_pass(g, dout):
        z = jnp.zeros((LANE, dout), jnp.float32)
        part = _make_sc_scatter(n_pad, groups, dout)(g, rowp, colp, z)
        return part[:, :n]

    def tc_mid(part, g, b, gw, gb, ga, w_next, dnext):
        return pl.pallas_call(
            _tc_mid_body,
            out_shape=jax.ShapeDtypeStruct((n, dnext), jnp.float32),
        )(part, g, dinv, _row(b), _row(gw), _row(gb), _row(ga), w_next)

    p1 = sc_pass(g1, dims[1])
    g2 = tc_mid(p1, g1, p['bs1'], p['gn1_w'], p['gn1_b'], p['gn1_a'],
                p['Ws2'], dims[2])
    p2 = sc_pass(g2, dims[2])
    g3 = tc_mid(p2, g2, p['bs2'], p['gn2_w'], p['gn2_b'], p['gn2_a'],
                p['Ws3'], dims[3])
    p3 = sc_pass(g3, dims[3])

    out_dim = p['Wl3'].shape[1]
    h, sig = pl.pallas_call(
        functools.partial(_tc_head_body, num_graphs=num_graphs),
        out_shape=(jax.ShapeDtypeStruct((num_graphs, out_dim), jnp.float32),
                   jax.ShapeDtypeStruct((num_graphs, out_dim), jnp.float32)),
    )(p3, g3, dinv, _row(p['bs3']), _row(p['gn3_w']), _row(p['gn3_b']),
      _row(p['gn3_a']), _row(xs_batch.astype(jnp.int32)),
      p['Wl1'], _row(p['bl1']), _row(p['bn1_w']), _row(p['bn1_b']),
      p['Wl2'], _row(p['bl2']), _row(p['bn2_w']), _row(p['bn2_b']),
      p['Wl3'], _row(p['bl3']))
    return (h, sig)


# speedup vs baseline: 42.0536x; 1.2335x over previous
import functools

import jax
import jax.numpy as jnp
from jax import lax
from jax.experimental import pallas as pl
from jax.experimental.pallas import tpu as pltpu
from jax.experimental.pallas import tpu_sc as plsc

NC = 2
NS = 16
NW = NC * NS
LANE = 128


def _sc_mesh():
    return plsc.VectorSubcoreMesh(core_axis_name="c", subcore_axis_name="s")


def _make_sc_degree(n_pad, groups):
    rpt = n_pad // NS

    @functools.partial(
        pl.kernel,
        out_type=jax.ShapeDtypeStruct((NC, n_pad, 1), jnp.float32),
        mesh=_sc_mesh(),
        scratch_types=[
            pltpu.VMEM((groups, LANE), jnp.int32),
            pltpu.VMEM((LANE, 1), jnp.float32),
            pltpu.VMEM((LANE, 1), jnp.float32),
            pltpu.VMEM_SHARED((n_pad, 1), jnp.float32),
        ],
        compiler_params=pltpu.CompilerParams(use_tc_tiling_on_sc=False),
    )
    def deg_kernel(col_hbm, ones_hbm, zero_hbm, out_hbm, cidx, ones_v, zbuf, acc):
        cid = lax.axis_index("c")
        sid = lax.axis_index("s")
        w = cid * NS + sid
        base = sid * rpt
        pltpu.sync_copy(ones_hbm, ones_v)
        pltpu.sync_copy(zero_hbm, zbuf)
        off = 0
        while off < rpt:
            sz = min(LANE, rpt - off)
            pltpu.sync_copy(zbuf.at[pl.ds(0, sz)], acc.at[pl.ds(base + off, sz)])
            off += sz
        plsc.subcore_barrier()
        pltpu.sync_copy(col_hbm.at[w], cidx)

        def body(g, carry):
            pltpu.sync_copy(ones_v, acc.at[cidx.at[g]], add=True)
            return carry

        lax.fori_loop(0, groups, body, 0)
        plsc.subcore_barrier()
        off = 0
        while off < rpt:
            sz = min(LANE, rpt - off)
            pltpu.sync_copy(acc.at[pl.ds(base + off, sz)],
                            out_hbm.at[cid, pl.ds(base + off, sz)])
            off += sz

    return deg_kernel


def _make_sc_scatter(n_pad, groups, dout):
    rpt = n_pad // NS

    @functools.partial(
        pl.kernel,
        out_type=jax.ShapeDtypeStruct((NC, n_pad, dout), jnp.float32),
        mesh=_sc_mesh(),
        scratch_types=[
            pltpu.VMEM((groups, LANE), jnp.int32),
            pltpu.VMEM((groups, LANE), jnp.int32),
            [pltpu.VMEM((LANE, dout), jnp.float32)] * 4,
            pltpu.VMEM((LANE, dout), jnp.float32),
            pltpu.VMEM_SHARED((n_pad, dout), jnp.float32),
            [pltpu.SemaphoreType.DMA] * 4,
            [pltpu.SemaphoreType.DMA] * 4,
            pltpu.SemaphoreType.DMA,
        ],
        compiler_params=pltpu.CompilerParams(use_tc_tiling_on_sc=False),
    )
    def scatter_kernel(g_hbm, row_hbm, col_hbm, zero_hbm, out_hbm,
                       ridx, cidx, bufs, zbuf, acc, gsem, ssem, psem):
        cid = lax.axis_index("c")
        sid = lax.axis_index("s")
        w = cid * NS + sid
        base = sid * rpt
        pltpu.async_copy(row_hbm.at[w], ridx, psem)
        pltpu.async_copy(col_hbm.at[w], cidx, psem)
        pltpu.sync_copy(zero_hbm, zbuf)
        off = 0
        while off < rpt:
            sz = min(LANE, rpt - off)
            pltpu.sync_copy(zbuf.at[pl.ds(0, sz)], acc.at[pl.ds(base + off, sz)])
            off += sz
        pltpu.make_async_copy(row_hbm.at[w], ridx, psem).wait()
        pltpu.make_async_copy(col_hbm.at[w], cidx, psem).wait()
        plsc.subcore_barrier()

        assert groups % 4 == 0
        quads = groups // 4
        for j in range(4):
            pltpu.async_copy(g_hbm.at[ridx.at[j]], bufs[j], gsem[j])

        def body(i, carry):
            g0 = 4 * i
            for j in range(4):
                pltpu.make_async_copy(g_hbm.at[ridx.at[g0 + j]], bufs[j],
                                      gsem[j]).wait()
                pltpu.async_copy(bufs[j], acc.at[cidx.at[g0 + j]], ssem[j],
                                 add=True)
            for j in range(4):
                pltpu.make_async_copy(bufs[j], acc.at[cidx.at[g0 + j]],
                                      ssem[j]).wait()
                pltpu.async_copy(g_hbm.at[ridx.at[g0 + 4 + j]], bufs[j],
                                 gsem[j])
            return carry

        lax.fori_loop(0, quads - 1, body, 0)
        g0 = groups - 4
        for j in range(4):
            pltpu.make_async_copy(g_hbm.at[ridx.at[g0 + j]], bufs[j],
                                  gsem[j]).wait()
            pltpu.async_copy(bufs[j], acc.at[cidx.at[g0 + j]], ssem[j],
                             add=True)
        for j in range(4):
            pltpu.make_async_copy(bufs[j], acc.at[cidx.at[g0 + j]],
                                  ssem[j]).wait()
        plsc.subcore_barrier()
        off = 0
        while off < rpt:
            sz = min(LANE, rpt - off)
            pltpu.sync_copy(acc.at[pl.ds(base + off, sz)],
                            out_hbm.at[cid, pl.ds(base + off, sz)])
            off += sz

    return scatter_kernel


def _tc_layer1_body(x_ref, w_ref, deg_ref, g_ref, dinv_ref):
    n = x_ref.shape[0]
    deg = 1.0 + deg_ref[0, :n] + deg_ref[1, :n]
    dinv = lax.rsqrt(deg)
    dinv_ref[...] = dinv
    h = jnp.dot(x_ref[...], w_ref[...], preferred_element_type=jnp.float32)
    g_ref[...] = dinv * h


def _tc_mid_body(p_ref, g_ref, dinv_ref, b_ref, gw_ref, gb_ref, ga_ref,
                 w_ref, out_ref):
    n = g_ref.shape[0]
    dinv = dinv_ref[...]
    s = p_ref[0, :n] + p_ref[1, :n] + g_ref[...]
    y = jnp.maximum(dinv * s + b_ref[...], 0.0)
    a = ga_ref[...]
    m = jnp.mean(y, axis=0, keepdims=True)
    o = y - a * m
    var = jnp.mean(o * o, axis=0, keepdims=True)
    xg = gw_ref[...] * o / jnp.sqrt(var + 1e-5) + gb_ref[...]
    h = jnp.dot(xg, w_ref[...], preferred_element_type=jnp.float32)
    out_ref[...] = dinv * h


def _tc_head_body(p_ref, g_ref, dinv_ref, b_ref, gw_ref, gb_ref, ga_ref,
                  batch_ref, wl1_ref, bl1_ref, bn1w_ref, bn1b_ref,
                  wl2_ref, bl2_ref, bn2w_ref, bn2b_ref, wl3_ref, bl3_ref,
                  h_ref, sig_ref, *, num_graphs):
    n = g_ref.shape[0]
    s = p_ref[0, :n] + p_ref[1, :n] + g_ref[...]
    y = jnp.maximum(dinv_ref[...] * s + b_ref[...], 0.0)
    a = ga_ref[...]
    m = jnp.mean(y, axis=0, keepdims=True)
    o = y - a * m
    var = jnp.mean(o * o, axis=0, keepdims=True)
    x3 = gw_ref[...] * o / jnp.sqrt(var + 1e-5) + gb_ref[...]

    n = x3.shape[0]
    gid = lax.broadcasted_iota(jnp.int32, (num_graphs, n), 0)
    oh = (gid == batch_ref[...]).astype(jnp.float32)
    ssum = jnp.dot(oh, x3, preferred_element_type=jnp.float32)
    cnt = jnp.sum(oh, axis=1, keepdims=True)
    xs = ssum / jnp.maximum(cnt, 1.0)

    h1 = jnp.dot(xs, wl1_ref[...], preferred_element_type=jnp.float32) + bl1_ref[...]
    m1 = jnp.mean(h1, axis=0, keepdims=True)
    v1 = jnp.mean((h1 - m1) ** 2, axis=0, keepdims=True)
    h1 = bn1w_ref[...] * (h1 - m1) / jnp.sqrt(v1 + 1e-5) + bn1b_ref[...]

    h2 = jnp.dot(h1, wl2_ref[...], preferred_element_type=jnp.float32) + bl2_ref[...]
    m2 = jnp.mean(h2, axis=0, keepdims=True)
    v2 = jnp.mean((h2 - m2) ** 2, axis=0, keepdims=True)
    h2 = bn2w_ref[...] * (h2 - m2) / jnp.sqrt(v2 + 1e-5) + bn2b_ref[...]

    h = jnp.dot(h2, wl3_ref[...], preferred_element_type=jnp.float32) + bl3_ref[...]
    h_ref[...] = h
    sig_ref[...] = jax.nn.sigmoid(h)


def _row(v):
    return v.reshape(1, -1)


def kernel(x_s, x_t, params, edge_index_s, edge_index_t, xs_batch, xt_batch):
    p = params
    n, d_in = x_s.shape
    e = edge_index_s.shape[1]
    num_graphs = 64

    assert e % NW == 0
    ept = e // NW
    groups = -(-ept // LANE)
    groups = ((groups + 3) // 4) * 4
    ept_pad = groups * LANE
    pad = ept_pad - ept
    n_pad = ((n + NS * 8 - 1) // (NS * 8)) * (NS * 8)
    if n_pad <= n + pad:
        n_pad += NS * 8

    row = edge_index_s[0].astype(jnp.int32).reshape(NW, ept)
    col = edge_index_s[1].astype(jnp.int32).reshape(NW, ept)
    pr = ((jnp.arange(pad, dtype=jnp.int32) * 89) % n)
    pc = n + (jnp.arange(pad, dtype=jnp.int32) % (n_pad - n))
    rowp = jnp.concatenate(
        [row, jnp.broadcast_to(pr, (NW, pad))], axis=1).reshape(NW, groups, LANE)
    colp = jnp.concatenate(
        [col, jnp.broadcast_to(pc, (NW, pad))], axis=1).reshape(NW, groups, LANE)

    deg2 = _make_sc_degree(n_pad, groups)(
        colp, jnp.ones((LANE, 1), jnp.float32), jnp.zeros((LANE, 1), jnp.float32))

    dims = [d_in, d_in // 2, d_in // 4, d_in // 8]

    g1, dinv = pl.pallas_call(
        _tc_layer1_body,
        out_shape=(jax.ShapeDtypeStruct((n, dims[1]), jnp.float32),
                   jax.ShapeDtypeStruct((n, 1), jnp.float32)),
    )(x_s, p['Ws1'], deg2)

    def sc_pass(g, dout):
        z = jnp.zeros((LANE, dout), jnp.float32)
        return _make_sc_scatter(n_pad, groups, dout)(g, rowp, colp, z)

    def tc_mid(part, g, b, gw, gb, ga, w_next, dnext):
        return pl.pallas_call(
            _tc_mid_body,
            out_shape=jax.ShapeDtypeStruct((n, dnext), jnp.float32),
        )(part, g, dinv, _row(b), _row(gw), _row(gb), _row(ga), w_next)

    p1 = sc_pass(g1, dims[1])
    g2 = tc_mid(p1, g1, p['bs1'], p['gn1_w'], p['gn1_b'], p['gn1_a'],
                p['Ws2'], dims[2])
    p2 = sc_pass(g2, dims[2])
    g3 = tc_mid(p2, g2, p['bs2'], p['gn2_w'], p['gn2_b'], p['gn2_a'],
                p['Ws3'], dims[3])
    p3 = sc_pass(g3, dims[3])

    out_dim = p['Wl3'].shape[1]
    h, sig = pl.pallas_call(
        functools.partial(_tc_head_body, num_graphs=num_graphs),
        out_shape=(jax.ShapeDtypeStruct((num_graphs, out_dim), jnp.float32),
                   jax.ShapeDtypeStruct((num_graphs, out_dim), jnp.float32)),
    )(p3, g3, dinv, _row(p['bs3']), _row(p['gn3_w']), _row(p['gn3_b']),
      _row(p['gn3_a']), _row(xs_batch.astype(jnp.int32)),
      p['Wl1'], _row(p['bl1']), _row(p['bn1_w']), _row(p['bn1_b']),
      p['Wl2'], _row(p['bl2']), _row(p['bn2_w']), _row(p['bn2_b']),
      p['Wl3'], _row(p['bl3']))
    return (h, sig)


# speedup vs baseline: 42.5602x; 1.0120x over previous
import functools

import jax
import jax.numpy as jnp
from jax import lax
from jax.experimental import pallas as pl
from jax.experimental.pallas import tpu as pltpu
from jax.experimental.pallas import tpu_sc as plsc

NC = 2
NS = 16
NW = NC * NS
LANE = 128


def _sc_mesh():
    return plsc.VectorSubcoreMesh(core_axis_name="c", subcore_axis_name="s")


def _make_sc_degree(n_pad, groups):
    rpt = n_pad // NS

    @functools.partial(
        pl.kernel,
        out_type=jax.ShapeDtypeStruct((NC, n_pad), jnp.float32),
        mesh=_sc_mesh(),
        scratch_types=[
            pltpu.VMEM((groups, LANE), jnp.int32),
            pltpu.VMEM((LANE,), jnp.float32),
            pltpu.VMEM((LANE,), jnp.float32),
            pltpu.VMEM_SHARED((n_pad,), jnp.float32),
        ],
    )
    def deg_kernel(col_hbm, ones_hbm, zero_hbm, out_hbm, cidx, ones_v, zbuf, acc):
        cid = lax.axis_index("c")
        sid = lax.axis_index("s")
        w = cid * NS + sid
        base = sid * rpt
        pltpu.sync_copy(ones_hbm, ones_v)
        pltpu.sync_copy(zero_hbm, zbuf)
        off = 0
        while off < rpt:
            sz = min(LANE, rpt - off)
            pltpu.sync_copy(zbuf.at[pl.ds(0, sz)], acc.at[pl.ds(base + off, sz)])
            off += sz
        plsc.subcore_barrier()
        pltpu.sync_copy(col_hbm.at[w], cidx)

        def body(g, carry):
            pltpu.sync_copy(ones_v, acc.at[cidx.at[g]], add=True)
            return carry

        lax.fori_loop(0, groups, body, 0)
        plsc.subcore_barrier()
        off = 0
        while off < rpt:
            sz = min(LANE, rpt - off)
            pltpu.sync_copy(acc.at[pl.ds(base + off, sz)],
                            out_hbm.at[cid, pl.ds(base + off, sz)])
            off += sz

    return deg_kernel


def _make_sc_scatter(n_pad, groups, dout):
    rpt = n_pad // NS

    @functools.partial(
        pl.kernel,
        out_type=jax.ShapeDtypeStruct((NC, n_pad, dout), jnp.float32),
        mesh=_sc_mesh(),
        scratch_types=[
            pltpu.VMEM((groups, LANE), jnp.int32),
            pltpu.VMEM((groups, LANE), jnp.int32),
            [pltpu.VMEM((LANE, dout), jnp.float32)] * 4,
            pltpu.VMEM((LANE, dout), jnp.float32),
            pltpu.VMEM_SHARED((n_pad, dout), jnp.float32),
            [pltpu.SemaphoreType.DMA] * 4,
            [pltpu.SemaphoreType.DMA] * 4,
            pltpu.SemaphoreType.DMA,
        ],
        compiler_params=pltpu.CompilerParams(use_tc_tiling_on_sc=False),
    )
    def scatter_kernel(g_hbm, row_hbm, col_hbm, zero_hbm, out_hbm,
                       ridx, cidx, bufs, zbuf, acc, gsem, ssem, psem):
        cid = lax.axis_index("c")
        sid = lax.axis_index("s")
        w = cid * NS + sid
        base = sid * rpt
        pltpu.async_copy(row_hbm.at[w], ridx, psem)
        pltpu.async_copy(col_hbm.at[w], cidx, psem)
        pltpu.sync_copy(zero_hbm, zbuf)
        off = 0
        while off < rpt:
            sz = min(LANE, rpt - off)
            pltpu.sync_copy(zbuf.at[pl.ds(0, sz)], acc.at[pl.ds(base + off, sz)])
            off += sz
        pltpu.make_async_copy(row_hbm.at[w], ridx, psem).wait()
        pltpu.make_async_copy(col_hbm.at[w], cidx, psem).wait()
        plsc.subcore_barrier()

        assert groups % 4 == 0
        quads = groups // 4
        for j in range(4):
            pltpu.async_copy(g_hbm.at[ridx.at[j]], bufs[j], gsem[j])

        def body(i, carry):
            g0 = 4 * i
            for j in range(4):
                pltpu.make_async_copy(g_hbm.at[ridx.at[g0 + j]], bufs[j],
                                      gsem[j]).wait()
                pltpu.async_copy(bufs[j], acc.at[cidx.at[g0 + j]], ssem[j],
                                 add=True)
            for j in range(4):
                pltpu.make_async_copy(bufs[j], acc.at[cidx.at[g0 + j]],
                                      ssem[j]).wait()
                pltpu.async_copy(g_hbm.at[ridx.at[g0 + 4 + j]], bufs[j],
                                 gsem[j])
            return carry

        lax.fori_loop(0, quads - 1, body, 0)
        g0 = groups - 4
        for j in range(4):
            pltpu.make_async_copy(g_hbm.at[ridx.at[g0 + j]], bufs[j],
                                  gsem[j]).wait()
            pltpu.async_copy(bufs[j], acc.at[cidx.at[g0 + j]], ssem[j],
                             add=True)
        for j in range(4):
            pltpu.make_async_copy(bufs[j], acc.at[cidx.at[g0 + j]],
                                  ssem[j]).wait()
        plsc.subcore_barrier()
        off = 0
        while off < rpt:
            sz = min(LANE, rpt - off)
            pltpu.sync_copy(acc.at[pl.ds(base + off, sz)],
                            out_hbm.at[cid, pl.ds(base + off, sz)])
            off += sz

    return scatter_kernel


def _tc_layer1_body(x_ref, w_ref, deg_ref, g_ref, dinv_ref):
    deg = 1.0 + deg_ref[0] + deg_ref[1]
    dinv = lax.rsqrt(deg)
    dinv_ref[...] = dinv
    h = jnp.dot(x_ref[...], w_ref[...], preferred_element_type=jnp.float32)
    g_ref[...] = dinv * h


def _tc_mid_body(p_ref, g_ref, dinv_ref, b_ref, gw_ref, gb_ref, ga_ref,
                 w_ref, out_ref):
    n = g_ref.shape[0]
    dinv = dinv_ref[...]
    s = p_ref[0, :n] + p_ref[1, :n] + g_ref[...]
    y = jnp.maximum(dinv * s + b_ref[...], 0.0)
    a = ga_ref[...]
    m = jnp.mean(y, axis=0, keepdims=True)
    o = y - a * m
    var = jnp.mean(o * o, axis=0, keepdims=True)
    xg = gw_ref[...] * o / jnp.sqrt(var + 1e-5) + gb_ref[...]
    h = jnp.dot(xg, w_ref[...], preferred_element_type=jnp.float32)
    out_ref[...] = dinv * h


def _tc_head_body(p_ref, g_ref, dinv_ref, b_ref, gw_ref, gb_ref, ga_ref,
                  batch_ref, wl1_ref, bl1_ref, bn1w_ref, bn1b_ref,
                  wl2_ref, bl2_ref, bn2w_ref, bn2b_ref, wl3_ref, bl3_ref,
                  h_ref, sig_ref, *, num_graphs):
    n = g_ref.shape[0]
    s = p_ref[0, :n] + p_ref[1, :n] + g_ref[...]
    y = jnp.maximum(dinv_ref[...] * s + b_ref[...], 0.0)
    a = ga_ref[...]
    m = jnp.mean(y, axis=0, keepdims=True)
    o = y - a * m
    var = jnp.mean(o * o, axis=0, keepdims=True)
    x3 = gw_ref[...] * o / jnp.sqrt(var + 1e-5) + gb_ref[...]

    n = x3.shape[0]
    gid = lax.broadcasted_iota(jnp.int32, (num_graphs, n), 0)
    oh = (gid == batch_ref[...]).astype(jnp.float32)
    ssum = jnp.dot(oh, x3, preferred_element_type=jnp.float32)
    cnt = jnp.sum(oh, axis=1, keepdims=True)
    xs = ssum / jnp.maximum(cnt, 1.0)

    h1 = jnp.dot(xs, wl1_ref[...], preferred_element_type=jnp.float32) + bl1_ref[...]
    m1 = jnp.mean(h1, axis=0, keepdims=True)
    v1 = jnp.mean((h1 - m1) ** 2, axis=0, keepdims=True)
    h1 = bn1w_ref[...] * (h1 - m1) / jnp.sqrt(v1 + 1e-5) + bn1b_ref[...]

    h2 = jnp.dot(h1, wl2_ref[...], preferred_element_type=jnp.float32) + bl2_ref[...]
    m2 = jnp.mean(h2, axis=0, keepdims=True)
    v2 = jnp.mean((h2 - m2) ** 2, axis=0, keepdims=True)
    h2 = bn2w_ref[...] * (h2 - m2) / jnp.sqrt(v2 + 1e-5) + bn2b_ref[...]

    h = jnp.dot(h2, wl3_ref[...], preferred_element_type=jnp.float32) + bl3_ref[...]
    h_ref[...] = h
    sig_ref[...] = jax.nn.sigmoid(h)


def _row(v):
    return v.reshape(1, -1)


def kernel(x_s, x_t, params, edge_index_s, edge_index_t, xs_batch, xt_batch):
    p = params
    n, d_in = x_s.shape
    e = edge_index_s.shape[1]
    num_graphs = 64

    assert e % NW == 0
    ept = e // NW
    groups = -(-ept // LANE)
    groups = ((groups + 3) // 4) * 4
    ept_pad = groups * LANE
    pad = ept_pad - ept
    n_pad = ((n + NS * 8 - 1) // (NS * 8)) * (NS * 8)
    if n_pad <= n + pad:
        n_pad += NS * 8

    row = edge_index_s[0].astype(jnp.int32).reshape(NW, ept)
    col = edge_index_s[1].astype(jnp.int32).reshape(NW, ept)
    pr = ((jnp.arange(pad, dtype=jnp.int32) * 89) % n)
    pc = n + (jnp.arange(pad, dtype=jnp.int32) % (n_pad - n))
    rowp = jnp.concatenate(
        [row, jnp.broadcast_to(pr, (NW, pad))], axis=1).reshape(NW, groups, LANE)
    colp = jnp.concatenate(
        [col, jnp.broadcast_to(pc, (NW, pad))], axis=1).reshape(NW, groups, LANE)

    deg2 = _make_sc_degree(n_pad, groups)(
        colp, jnp.ones((LANE,), jnp.float32), jnp.zeros((LANE,), jnp.float32))
    deg2 = deg2[:, :n].reshape(NC, n, 1)

    dims = [d_in, d_in // 2, d_in // 4, d_in // 8]

    g1, dinv = pl.pallas_call(
        _tc_layer1_body,
        out_shape=(jax.ShapeDtypeStruct((n, dims[1]), jnp.float32),
                   jax.ShapeDtypeStruct((n, 1), jnp.float32)),
    )(x_s, p['Ws1'], deg2)

    def sc_pass(g, dout):
        z = jnp.zeros((LANE, dout), jnp.float32)
        return _make_sc_scatter(n_pad, groups, dout)(g, rowp, colp, z)

    def tc_mid(part, g, b, gw, gb, ga, w_next, dnext):
        return pl.pallas_call(
            _tc_mid_body,
            out_shape=jax.ShapeDtypeStruct((n, dnext), jnp.float32),
        )(part, g, dinv, _row(b), _row(gw), _row(gb), _row(ga), w_next)

    p1 = sc_pass(g1, dims[1])
    g2 = tc_mid(p1, g1, p['bs1'], p['gn1_w'], p['gn1_b'], p['gn1_a'],
                p['Ws2'], dims[2])
    p2 = sc_pass(g2, dims[2])
    g3 = tc_mid(p2, g2, p['bs2'], p['gn2_w'], p['gn2_b'], p['gn2_a'],
                p['Ws3'], dims[3])
    p3 = sc_pass(g3, dims[3])

    out_dim = p['Wl3'].shape[1]
    h, sig = pl.pallas_call(
        functools.partial(_tc_head_body, num_graphs=num_graphs),
        out_shape=(jax.ShapeDtypeStruct((num_graphs, out_dim), jnp.float32),
                   jax.ShapeDtypeStruct((num_graphs, out_dim), jnp.float32)),
    )(p3, g3, dinv, _row(p['bs3']), _row(p['gn3_w']), _row(p['gn3_b']),
      _row(p['gn3_a']), _row(xs_batch.astype(jnp.int32)),
      p['Wl1'], _row(p['bl1']), _row(p['bn1_w']), _row(p['bn1_b']),
      p['Wl2'], _row(p['bl2']), _row(p['bn2_w']), _row(p['bn2_b']),
      p['Wl3'], _row(p['bl3']))
    return (h, sig)


# speedup vs baseline: 46.2981x; 1.0878x over previous
import functools

import jax
import jax.numpy as jnp
from jax import lax
from jax.experimental import pallas as pl
from jax.experimental.pallas import tpu as pltpu
from jax.experimental.pallas import tpu_sc as plsc

NC = 2
NS = 16
NW = NC * NS
LANE = 128


def _sc_mesh():
    return plsc.VectorSubcoreMesh(core_axis_name="c", subcore_axis_name="s")


def _make_sc_degree(n_pad, groups):
    rpt = n_pad // NS

    @functools.partial(
        pl.kernel,
        out_type=jax.ShapeDtypeStruct((NC, n_pad), jnp.float32),
        mesh=_sc_mesh(),
        scratch_types=[
            pltpu.VMEM((groups, LANE), jnp.int32),
            pltpu.VMEM((LANE,), jnp.float32),
            pltpu.VMEM((LANE,), jnp.float32),
            pltpu.VMEM_SHARED((n_pad,), jnp.float32),
        ],
    )
    def deg_kernel(col_hbm, ones_hbm, zero_hbm, out_hbm, cidx, ones_v, zbuf, acc):
        cid = lax.axis_index("c")
        sid = lax.axis_index("s")
        w = cid * NS + sid
        base = sid * rpt
        pltpu.sync_copy(ones_hbm, ones_v)
        pltpu.sync_copy(zero_hbm, zbuf)
        off = 0
        while off < rpt:
            sz = min(LANE, rpt - off)
            pltpu.sync_copy(zbuf.at[pl.ds(0, sz)], acc.at[pl.ds(base + off, sz)])
            off += sz
        plsc.subcore_barrier()
        pltpu.sync_copy(col_hbm.at[w], cidx)

        def body(g, carry):
            pltpu.sync_copy(ones_v, acc.at[cidx.at[g]], add=True)
            return carry

        lax.fori_loop(0, groups, body, 0)
        plsc.subcore_barrier()
        off = 0
        while off < rpt:
            sz = min(LANE, rpt - off)
            pltpu.sync_copy(acc.at[pl.ds(base + off, sz)],
                            out_hbm.at[cid, pl.ds(base + off, sz)])
            off += sz

    return deg_kernel


def _make_sc_scatter(n_pad, groups, dout):
    rpt = n_pad // NS

    @functools.partial(
        pl.kernel,
        out_type=jax.ShapeDtypeStruct((NC, n_pad, 128), jnp.float32),
        mesh=_sc_mesh(),
        scratch_types=[
            pltpu.VMEM((groups, LANE), jnp.int32),
            pltpu.VMEM((groups, LANE), jnp.int32),
            [pltpu.VMEM((LANE, dout), jnp.float32)] * 4,
            pltpu.VMEM((LANE, dout), jnp.float32),
            pltpu.VMEM_SHARED((n_pad, dout), jnp.float32),
            [pltpu.SemaphoreType.DMA] * 4,
            [pltpu.SemaphoreType.DMA] * 4,
            pltpu.SemaphoreType.DMA,
        ],
        compiler_params=pltpu.CompilerParams(use_tc_tiling_on_sc=False),
    )
    def scatter_kernel(g_hbm, row_hbm, col_hbm, zero_hbm, out_hbm,
                       ridx, cidx, bufs, zbuf, acc, gsem, ssem, psem):
        cid = lax.axis_index("c")
        sid = lax.axis_index("s")
        w = cid * NS + sid
        base = sid * rpt
        pltpu.async_copy(row_hbm.at[w], ridx, psem)
        pltpu.async_copy(col_hbm.at[w], cidx, psem)
        pltpu.sync_copy(zero_hbm, zbuf)
        off = 0
        while off < rpt:
            sz = min(LANE, rpt - off)
            pltpu.sync_copy(zbuf.at[pl.ds(0, sz)], acc.at[pl.ds(base + off, sz)])
            off += sz
        pltpu.make_async_copy(row_hbm.at[w], ridx, psem).wait()
        pltpu.make_async_copy(col_hbm.at[w], cidx, psem).wait()
        plsc.subcore_barrier()

        assert groups % 4 == 0
        quads = groups // 4
        for j in range(4):
            pltpu.async_copy(g_hbm.at[ridx.at[j]], bufs[j], gsem[j])

        def body(i, carry):
            g0 = 4 * i
            for j in range(4):
                pltpu.make_async_copy(g_hbm.at[ridx.at[g0 + j]], bufs[j],
                                      gsem[j]).wait()
                pltpu.async_copy(bufs[j], acc.at[cidx.at[g0 + j]], ssem[j],
                                 add=True)
            for j in range(4):
                pltpu.make_async_copy(bufs[j], acc.at[cidx.at[g0 + j]],
                                      ssem[j]).wait()
                pltpu.async_copy(g_hbm.at[ridx.at[g0 + 4 + j]], bufs[j],
                                 gsem[j])
            return carry

        lax.fori_loop(0, quads - 1, body, 0)
        g0 = groups - 4
        for j in range(4):
            pltpu.make_async_copy(g_hbm.at[ridx.at[g0 + j]], bufs[j],
                                  gsem[j]).wait()
            pltpu.async_copy(bufs[j], acc.at[cidx.at[g0 + j]], ssem[j],
                             add=True)
        for j in range(4):
            pltpu.make_async_copy(bufs[j], acc.at[cidx.at[g0 + j]],
                                  ssem[j]).wait()
        plsc.subcore_barrier()
        off = 0
        while off < rpt:
            sz = min(LANE, rpt - off)
            pltpu.sync_copy(acc.at[pl.ds(base + off, sz)],
                            out_hbm.at[cid, pl.ds(base + off, sz), pl.ds(0, dout)])
            off += sz

    return scatter_kernel


def _tc_layer1_body(x_ref, w_ref, deg_ref, g_ref, dinv_ref):
    deg = 1.0 + deg_ref[0] + deg_ref[1]
    dinv = lax.rsqrt(deg)
    dinv_ref[...] = dinv
    h = jnp.dot(x_ref[...], w_ref[...], preferred_element_type=jnp.float32)
    g_ref[...] = dinv * h


def _tc_mid_body(p_ref, g_ref, dinv_ref, b_ref, gw_ref, gb_ref, ga_ref,
                 w_ref, out_ref):
    n, dout = g_ref.shape
    dinv = dinv_ref[...]
    s = p_ref[0, :n, :dout] + p_ref[1, :n, :dout] + g_ref[...]
    y = jnp.maximum(dinv * s + b_ref[...], 0.0)
    a = ga_ref[...]
    m = jnp.mean(y, axis=0, keepdims=True)
    o = y - a * m
    var = jnp.mean(o * o, axis=0, keepdims=True)
    xg = gw_ref[...] * o / jnp.sqrt(var + 1e-5) + gb_ref[...]
    h = jnp.dot(xg, w_ref[...], preferred_element_type=jnp.float32)
    out_ref[...] = dinv * h


def _tc_head_body(p_ref, g_ref, dinv_ref, b_ref, gw_ref, gb_ref, ga_ref,
                  batch_ref, wl1_ref, bl1_ref, bn1w_ref, bn1b_ref,
                  wl2_ref, bl2_ref, bn2w_ref, bn2b_ref, wl3_ref, bl3_ref,
                  h_ref, sig_ref, *, num_graphs):
    n = g_ref.shape[0]
    dout = g_ref.shape[1]
    s = p_ref[0, :n, :dout] + p_ref[1, :n, :dout] + g_ref[...]
    y = jnp.maximum(dinv_ref[...] * s + b_ref[...], 0.0)
    a = ga_ref[...]
    m = jnp.mean(y, axis=0, keepdims=True)
    o = y - a * m
    var = jnp.mean(o * o, axis=0, keepdims=True)
    x3 = gw_ref[...] * o / jnp.sqrt(var + 1e-5) + gb_ref[...]

    n = x3.shape[0]
    gid = lax.broadcasted_iota(jnp.int32, (num_graphs, n), 0)
    oh = (gid == batch_ref[...]).astype(jnp.float32)
    ssum = jnp.dot(oh, x3, preferred_element_type=jnp.float32)
    cnt = jnp.sum(oh, axis=1, keepdims=True)
    xs = ssum / jnp.maximum(cnt, 1.0)

    h1 = jnp.dot(xs, wl1_ref[...], preferred_element_type=jnp.float32) + bl1_ref[...]
    m1 = jnp.mean(h1, axis=0, keepdims=True)
    v1 = jnp.mean((h1 - m1) ** 2, axis=0, keepdims=True)
    h1 = bn1w_ref[...] * (h1 - m1) / jnp.sqrt(v1 + 1e-5) + bn1b_ref[...]

    h2 = jnp.dot(h1, wl2_ref[...], preferred_element_type=jnp.float32) + bl2_ref[...]
    m2 = jnp.mean(h2, axis=0, keepdims=True)
    v2 = jnp.mean((h2 - m2) ** 2, axis=0, keepdims=True)
    h2 = bn2w_ref[...] * (h2 - m2) / jnp.sqrt(v2 + 1e-5) + bn2b_ref[...]

    h = jnp.dot(h2, wl3_ref[...], preferred_element_type=jnp.float32) + bl3_ref[...]
    h_ref[...] = h
    sig_ref[...] = jax.nn.sigmoid(h)


def _row(v):
    return v.reshape(1, -1)


def kernel(x_s, x_t, params, edge_index_s, edge_index_t, xs_batch, xt_batch):
    p = params
    n, d_in = x_s.shape
    e = edge_index_s.shape[1]
    num_graphs = 64

    assert e % NW == 0
    ept = e // NW
    groups = -(-ept // LANE)
    groups = ((groups + 3) // 4) * 4
    ept_pad = groups * LANE
    pad = ept_pad - ept
    n_pad = ((n + NS * 8 - 1) // (NS * 8)) * (NS * 8)
    if n_pad <= n + pad:
        n_pad += NS * 8

    row = edge_index_s[0].astype(jnp.int32).reshape(NW, ept)
    col = edge_index_s[1].astype(jnp.int32).reshape(NW, ept)
    pr = ((jnp.arange(pad, dtype=jnp.int32) * 89) % n)
    pc = n + (jnp.arange(pad, dtype=jnp.int32) % (n_pad - n))
    rowp = jnp.concatenate(
        [row, jnp.broadcast_to(pr, (NW, pad))], axis=1).reshape(NW, groups, LANE)
    colp = jnp.concatenate(
        [col, jnp.broadcast_to(pc, (NW, pad))], axis=1).reshape(NW, groups, LANE)

    deg2 = _make_sc_degree(n_pad, groups)(
        colp, jnp.ones((LANE,), jnp.float32), jnp.zeros((LANE,), jnp.float32))
    deg2 = deg2[:, :n].reshape(NC, n, 1)

    dims = [d_in, d_in // 2, d_in // 4, d_in // 8]

    g1, dinv = pl.pallas_call(
        _tc_layer1_body,
        out_shape=(jax.ShapeDtypeStruct((n, dims[1]), jnp.float32),
                   jax.ShapeDtypeStruct((n, 1), jnp.float32)),
    )(x_s, p['Ws1'], deg2)

    def sc_pass(g, dout):
        z = jnp.zeros((LANE, dout), jnp.float32)
        return _make_sc_scatter(n_pad, groups, dout)(g, rowp, colp, z)

    def tc_mid(part, g, b, gw, gb, ga, w_next, dnext):
        return pl.pallas_call(
            _tc_mid_body,
            out_shape=jax.ShapeDtypeStruct((n, dnext), jnp.float32),
        )(part, g, dinv, _row(b), _row(gw), _row(gb), _row(ga), w_next)

    p1 = sc_pass(g1, dims[1])
    g2 = tc_mid(p1, g1, p['bs1'], p['gn1_w'], p['gn1_b'], p['gn1_a'],
                p['Ws2'], dims[2])
    p2 = sc_pass(g2, dims[2])
    g3 = tc_mid(p2, g2, p['bs2'], p['gn2_w'], p['gn2_b'], p['gn2_a'],
                p['Ws3'], dims[3])
    p3 = sc_pass(g3, dims[3])

    out_dim = p['Wl3'].shape[1]
    h, sig = pl.pallas_call(
        functools.partial(_tc_head_body, num_graphs=num_graphs),
        out_shape=(jax.ShapeDtypeStruct((num_graphs, out_dim), jnp.float32),
                   jax.ShapeDtypeStruct((num_graphs, out_dim), jnp.float32)),
    )(p3, g3, dinv, _row(p['bs3']), _row(p['gn3_w']), _row(p['gn3_b']),
      _row(p['gn3_a']), _row(xs_batch.astype(jnp.int32)),
      p['Wl1'], _row(p['bl1']), _row(p['bn1_w']), _row(p['bn1_b']),
      p['Wl2'], _row(p['bl2']), _row(p['bn2_w']), _row(p['bn2_b']),
      p['Wl3'], _row(p['bl3']))
    return (h, sig)
